# Initial kernel scaffold; baseline (speedup 1.0000x reference)
import jax
import jax.numpy as jnp
from jax.experimental import pallas as pl


def kernel(pos, batch, W1, b1, W2, b2, W3, b3, W4, b4, W5, b5):
    raise NotImplementedError("write your pallas kernel here")


# speedup vs baseline: 14.6463x; 14.6463x over previous
import functools

import jax
import jax.numpy as jnp
from jax import lax
from jax.experimental import pallas as pl
from jax.experimental.pallas import tpu as pltpu, tpu_sc as plsc

B = 32
P = 1024
N = B * P
H = 128
K = 16
f32 = jnp.float32

_BIG = 1e10


def _knn_body(pc_ref, pr_ref, idx_ref, rx_ref, ry_ref):
    b = pl.program_id(0)
    xi = pc_ref[0, :, 0:1]
    yi = pc_ref[0, :, 1:2]
    xj = pr_ref[0, 0:1, :]
    yj = pr_ref[0, 1:2, :]
    dx = xi - xj
    dy = yi - yj
    d = dx * dx + dy * dy
    ci = lax.broadcasted_iota(jnp.int32, (P, P), 1)
    ri = lax.broadcasted_iota(jnp.int32, (P, P), 0)
    d = jnp.where(ci == ri, _BIG, d)
    cols, rxs, rys = [], [], []
    for _ in range(K):
        m = jnp.min(d, axis=1, keepdims=True)
        t = jnp.where(d == m, ci, P)
        jm = jnp.min(t, axis=1, keepdims=True)
        cols.append(jm)
        sel = ci == jm
        rxs.append(-jnp.sum(jnp.where(sel, dx, 0.0), axis=1, keepdims=True))
        rys.append(-jnp.sum(jnp.where(sel, dy, 0.0), axis=1, keepdims=True))
        d = jnp.where(sel, _BIG, d)
    idx_ref[0] = jnp.concatenate(cols, axis=1) + b * P
    rx_ref[0] = jnp.concatenate(rxs, axis=1).astype(jnp.bfloat16)
    ry_ref[0] = jnp.concatenate(rys, axis=1).astype(jnp.bfloat16)


def _knn(pos3, pos3t):
    return pl.pallas_call(
        _knn_body,
        grid=(B,),
        in_specs=[
            pl.BlockSpec((1, P, 2), lambda b: (b, 0, 0)),
            pl.BlockSpec((1, 2, P), lambda b: (b, 0, 0)),
        ],
        out_specs=[pl.BlockSpec((1, P, K), lambda b: (b, 0, 0))] * 3,
        out_shape=[
            jax.ShapeDtypeStruct((B, P, K), jnp.int32),
            jax.ShapeDtypeStruct((B, P, K), jnp.bfloat16),
            jax.ShapeDtypeStruct((B, P, K), jnp.bfloat16),
        ],
    )(pos3, pos3t)


def _conv1_body(rx_ref, ry_ref, w_ref, b_ref, h_ref, cs_ref, ss_ref):
    w0 = w_ref[0:1, :]
    w1 = w_ref[1:2, :]
    acc = None
    for k in range(K):
        rx = rx_ref[0, :, k:k + 1].astype(f32)
        ry = ry_ref[0, :, k:k + 1].astype(f32)
        m = rx * w0 + ry * w1
        acc = m if acc is None else jnp.maximum(acc, m)
    h = jnp.maximum(acc + b_ref[...], 0.0)
    h_ref[0] = h
    cs_ref[0] = jnp.sum(h, axis=0, keepdims=True)
    ss_ref[0] = jnp.sum(h * h, axis=0, keepdims=True)


def _conv1(rx, ry, w1b, bias):
    return pl.pallas_call(
        _conv1_body,
        grid=(B,),
        in_specs=[
            pl.BlockSpec((1, P, K), lambda b: (b, 0, 0)),
            pl.BlockSpec((1, P, K), lambda b: (b, 0, 0)),
            pl.BlockSpec((2, H), lambda b: (0, 0)),
            pl.BlockSpec((1, H), lambda b: (0, 0)),
        ],
        out_specs=[
            pl.BlockSpec((1, P, H), lambda b: (b, 0, 0)),
            pl.BlockSpec((1, 1, H), lambda b: (b, 0, 0)),
            pl.BlockSpec((1, 1, H), lambda b: (b, 0, 0)),
        ],
        out_shape=[
            jax.ShapeDtypeStruct((B, P, H), f32),
            jax.ShapeDtypeStruct((B, 1, H), f32),
            jax.ShapeDtypeStruct((B, 1, H), f32),
        ],
    )(rx, ry, w1b, bias)


_BLK = 2048
_NB = N // _BLK


def _pre_body(pos_ref, w_ref, c2_ref, c3_ref):
    r = jnp.dot(pos_ref[...], w_ref[...], preferred_element_type=f32,
                precision=lax.Precision.HIGHEST)
    c2_ref[...] = r[:, :H]
    c3_ref[...] = r[:, H:]


def _pre(pos, wcat):
    outs = [jax.ShapeDtypeStruct((N, H), f32)] * 2
    return pl.pallas_call(
        _pre_body,
        grid=(_NB,),
        in_specs=[
            pl.BlockSpec((_BLK, 2), lambda i: (i, 0)),
            pl.BlockSpec((2, 2 * H), lambda i: (0, 0)),
        ],
        out_specs=[pl.BlockSpec((_BLK, H), lambda i: (i, 0))] * 2,
        out_shape=outs,
    )(pos, wcat)


def _hstat_body(g_ref, b_ref, h_ref, cs_ref, ss_ref):
    h = jnp.maximum(g_ref[...] + b_ref[...], 0.0)
    h_ref[...] = h
    cs_ref[0] = jnp.sum(h, axis=0, keepdims=True)
    ss_ref[0] = jnp.sum(h * h, axis=0, keepdims=True)


def _hstat(g, bias):
    return pl.pallas_call(
        _hstat_body,
        grid=(_NB,),
        in_specs=[
            pl.BlockSpec((_BLK, H), lambda i: (i, 0)),
            pl.BlockSpec((1, H), lambda i: (0, 0)),
        ],
        out_specs=[
            pl.BlockSpec((_BLK, H), lambda i: (i, 0)),
            pl.BlockSpec((1, 1, H), lambda i: (i, 0, 0)),
            pl.BlockSpec((1, 1, H), lambda i: (i, 0, 0)),
        ],
        out_shape=[
            jax.ShapeDtypeStruct((N, H), f32),
            jax.ShapeDtypeStruct((_NB, 1, H), f32),
            jax.ShapeDtypeStruct((_NB, 1, H), f32),
        ],
    )(g, bias)


def _pn_scale(cs, ss):
    mu = jnp.sum(cs, axis=(0, 1), keepdims=False)[None] * (1.0 / N)
    sst = jnp.sum(ss)
    musq = jnp.sum(mu * mu)
    scale = 1.0 / jnp.sqrt(1e-5 + sst * (1.0 / N) - musq)
    return mu, scale


def _nmm_body(h_ref, cs_ref, ss_ref, w_ref, z_ref):
    mu, scale = _pn_scale(cs_ref[...], ss_ref[...])
    hn = (h_ref[...] - mu) * scale
    z_ref[...] = jnp.dot(hn.astype(jnp.bfloat16), w_ref[...].astype(jnp.bfloat16),
                         preferred_element_type=f32)


def _nmm(h, cs, ss, w):
    npart = cs.shape[0]
    return pl.pallas_call(
        _nmm_body,
        grid=(_NB,),
        in_specs=[
            pl.BlockSpec((_BLK, H), lambda i: (i, 0)),
            pl.BlockSpec((npart, 1, H), lambda i: (0, 0, 0)),
            pl.BlockSpec((npart, 1, H), lambda i: (0, 0, 0)),
            pl.BlockSpec((H, H), lambda i: (0, 0)),
        ],
        out_specs=pl.BlockSpec((_BLK, H), lambda i: (i, 0)),
        out_shape=jax.ShapeDtypeStruct((N, H), f32),
    )(h, cs, ss, w)


def _final_body(h_ref, cs_ref, ss_ref, w4_ref, b4_ref, w5_ref, b5_ref, o_ref):
    mu, scale = _pn_scale(cs_ref[...], ss_ref[...])
    hn = (h_ref[...] - mu) * scale
    pooled = jnp.max(hn, axis=0, keepdims=True)
    r = jnp.maximum(
        jnp.dot(pooled.astype(jnp.bfloat16), w4_ref[...].astype(jnp.bfloat16),
                preferred_element_type=f32) + b4_ref[...], 0.0)
    o = jnp.dot(r.astype(jnp.bfloat16), w5_ref[...].astype(jnp.bfloat16),
                preferred_element_type=f32) + b5_ref[...]
    o_ref[0] = o


def _final(h3, cs, ss, w4, b4, w5, b5):
    return pl.pallas_call(
        _final_body,
        grid=(B,),
        in_specs=[
            pl.BlockSpec((P, H), lambda b: (b, 0)),
            pl.BlockSpec((_NB, 1, H), lambda b: (0, 0, 0)),
            pl.BlockSpec((_NB, 1, H), lambda b: (0, 0, 0)),
            pl.BlockSpec((H, H), lambda b: (0, 0)),
            pl.BlockSpec((1, H), lambda b: (0, 0)),
            pl.BlockSpec((H, 2), lambda b: (0, 0)),
            pl.BlockSpec((1, 2), lambda b: (0, 0)),
        ],
        out_specs=pl.BlockSpec((1, 1, 2), lambda b: (b, 0, 0)),
        out_shape=jax.ShapeDtypeStruct((B, 1, 2), f32),
    )(h3, cs, ss, w4, b4, w5, b5)


_NCHUNK = P * K // 128
_CPB = 16


def _convn_body(a_hbm, idx_hbm, rx_hbm, ry_hbm, wb_hbm, out_hbm,
                idx_v, rx_v, ry_v, wb_v, gbuf, obuf, gsem):
    w = lax.axis_index("s") * 2 + lax.axis_index("c")
    base = w * (P * K)
    pltpu.sync_copy(idx_hbm.at[pl.ds(base, P * K)], idx_v)
    pltpu.sync_copy(rx_hbm.at[pl.ds(base, P * K)], rx_v)
    pltpu.sync_copy(ry_hbm.at[pl.ds(base, P * K)], ry_v)
    pltpu.sync_copy(wb_hbm, wb_v)
    w0 = [wb_v[pl.ds(s * 16, 16)] for s in range(8)]
    w1 = [wb_v[pl.ds(H + s * 16, 16)] for s in range(8)]

    def blk_body(blk, _):
        def chunk_body(c, _):
            r = blk * _CPB + c
            pltpu.async_copy(a_hbm.at[idx_v.at[pl.ds(r * 128, 128)]], gbuf, gsem).wait()
            for p in range(8):
                e0 = r * 128 + p * K
                vsx = rx_v[pl.ds(e0, K)]
                vsy = ry_v[pl.ds(e0, K)]
                accs = [None] * 8
                for k in range(K):
                    sx = vsx[k]
                    sy = vsy[k]
                    for s in range(8):
                        t = gbuf[p * K + k, pl.ds(s * 16, 16)] + (sx * w0[s] + sy * w1[s])
                        accs[s] = t if k == 0 else jnp.maximum(accs[s], t)
                for s in range(8):
                    obuf[c * 8 + p, pl.ds(s * 16, 16)] = accs[s]
            return _
        lax.fori_loop(0, _CPB, chunk_body, None)
        pltpu.sync_copy(obuf, out_hbm.at[pl.ds(w * P + blk * 128, 128)])
        return _
    lax.fori_loop(0, P // 128, blk_body, None)


@functools.cache
def _convn_fn():
    mesh = plsc.VectorSubcoreMesh(core_axis_name="c", subcore_axis_name="s")
    return pl.kernel(
        _convn_body,
        out_type=jax.ShapeDtypeStruct((N, H), f32),
        mesh=mesh,
        scratch_types=[
            pltpu.VMEM((P * K,), jnp.int32),
            pltpu.VMEM((P * K,), f32),
            pltpu.VMEM((P * K,), f32),
            pltpu.VMEM((2 * H,), f32),
            pltpu.VMEM((128, H), f32),
            pltpu.VMEM((128, H), f32),
            pltpu.SemaphoreType.DMA,
        ],
    )


def _convn(a, idx1, rxf, ryf, wb):
    return _convn_fn()(a, idx1, rxf, ryf, wb)


def _round_bf16(x):
    b = jax.lax.bitcast_convert_type(x, jnp.int32)
    r = b + jnp.int32(0x7FFF) + ((b >> 16) & 1)
    r = r & jnp.int32(-65536)
    return jax.lax.bitcast_convert_type(r, f32)


def kernel(pos, batch, W1, b1, W2, b2, W3, b3, W4, b4, W5, b5):
    pos3 = pos.reshape(B, P, 2)
    pos3t = pos3.transpose(0, 2, 1)
    idx, rx, ry = _knn(pos3, pos3t)
    idx1 = idx.reshape(B * P * K)
    rxf = rx.astype(f32).reshape(B * P * K)
    ryf = ry.astype(f32).reshape(B * P * K)

    w1b = _round_bf16(W1)
    w2b = _round_bf16(W2[H:]).reshape(2 * H)
    w3b = _round_bf16(W3[H:]).reshape(2 * H)

    h1, cs1, ss1 = _conv1(rx, ry, w1b, b1[None])
    z2 = _nmm(h1.reshape(N, H), cs1, ss1, W2[:H])

    g2 = _convn(z2, idx1, rxf, ryf, w2b)
    g2, _, _, _, _, _ = lax.optimization_barrier((g2, z2, idx1, rxf, ryf, w2b))
    h2, cs2, ss2 = _hstat(g2, b2[None])
    z3 = _nmm(h2, cs2, ss2, W3[:H])

    g3 = _convn(z3, idx1, rxf, ryf, w3b)
    g3, _, _, _, _, _ = lax.optimization_barrier((g3, z3, idx1, rxf, ryf, w3b))
    h3, cs3, ss3 = _hstat(g3, b3[None])

    out = _final(h3, cs3, ss3, W4, b4[None], W5, b5[None])
    return out.reshape(B, 2)
